# baseline (speedup 1.0000x reference)
import jax
import jax.numpy as jnp
import numpy as np
from jax.experimental import pallas as pl
from jax.experimental.pallas import tpu as pltpu

_SCORE_THR = 0.05
_IOU_THR = 0.5
_IMTOP = 100
_CANVAS_H = 800.0
_CANVAS_W = 1333.0
_CLIP = float(np.log(1000.0 / 16.0))

_BR = 128
_BC = 128


def _prep_body(inp_ref, stats_ref, boxes_ref):
    dx = inp_ref[0:1, :] * 0.1
    dy = inp_ref[1:2, :] * 0.1
    dw = jnp.minimum(inp_ref[2:3, :] * 0.2, _CLIP)
    dh = jnp.minimum(inp_ref[3:4, :] * 0.2, _CLIP)
    pw = inp_ref[6:7, :]
    ph = inp_ref[7:8, :]
    cx = inp_ref[4:5, :] + dx * pw
    cy = inp_ref[5:6, :] + dy * ph
    w = pw * jnp.exp(dw)
    h = ph * jnp.exp(dh)
    x1 = jnp.clip(cx - 0.5 * w, 0.0, _CANVAS_W)
    y1 = jnp.clip(cy - 0.5 * h, 0.0, _CANVAS_H)
    x2 = jnp.clip(cx + 0.5 * w, 0.0, _CANVAS_W)
    y2 = jnp.clip(cy + 0.5 * h, 0.0, _CANVAS_H)
    bw = x2 - x1
    bh = y2 - y1
    s = inp_ref[8:9, :]
    valid = (bw > 0.0) & (bh > 0.0) & (s > _SCORE_THR)
    off = inp_ref[9:10, :] * (_CANVAS_W + 1.0)
    stats_ref[0:1, :] = x1 + off
    stats_ref[1:2, :] = y1
    stats_ref[2:3, :] = x2 + off
    stats_ref[3:4, :] = y2
    stats_ref[4:5, :] = bw * bh
    stats_ref[5:6, :] = s
    stats_ref[6:7, :] = valid.astype(jnp.float32)
    stats_ref[7:8, :] = inp_ref[10:11, :]
    boxes_ref[0:1, :] = x1
    boxes_ref[1:2, :] = y1
    boxes_ref[2:3, :] = x2
    boxes_ref[3:4, :] = y2


def _sup_body(jinfo_ref, statsI_ref, stats3_ref, out_ref):
    i = pl.program_id(0)
    j0 = jinfo_ref[2 * i]
    jn = jinfo_ref[2 * i + 1]
    x1i = statsI_ref[:, 0:1]
    y1i = statsI_ref[:, 1:2]
    x2i = statsI_ref[:, 2:3]
    y2i = statsI_ref[:, 3:4]
    ai = statsI_ref[:, 4:5]
    si = statsI_ref[:, 5:6]
    ii = statsI_ref[:, 7:8]

    def body(k, acc):
        blk = stats3_ref[j0 + k]
        x1j = blk[0:1, :]
        y1j = blk[1:2, :]
        x2j = blk[2:3, :]
        y2j = blk[3:4, :]
        aj = blk[4:5, :]
        sj = blk[5:6, :]
        vj = blk[6:7, :]
        ij = blk[7:8, :]
        iw = jnp.maximum(jnp.minimum(x2i, x2j) - jnp.maximum(x1i, x1j), 0.0)
        ih = jnp.maximum(jnp.minimum(y2i, y2j) - jnp.maximum(y1i, y1j), 0.0)
        inter = iw * ih
        iou = inter / (ai + aj - inter + 1e-9)
        prec = (sj > si) | ((sj == si) & (ij < ii))
        hit = (iou > _IOU_THR) & prec & (vj > 0.5)
        sup = jnp.any(hit, axis=1, keepdims=True).astype(jnp.float32)
        return jnp.maximum(acc, sup)

    acc = jax.lax.fori_loop(0, jn, body, jnp.zeros((_BR, 1), jnp.float32))
    out_ref[...] = jnp.broadcast_to(acc, out_ref.shape)


def kernel(reg, proposals, scores, classes):
    n = reg.shape[0]
    npad = ((n + _BC - 1) // _BC) * _BC
    pad = npad - n
    n_i = npad // _BR
    n_j = npad // _BC

    cls_pad = jnp.concatenate([classes.astype(jnp.int32), jnp.full((pad,), 100, jnp.int32)])
    perm = jnp.argsort(cls_pad)
    csort = cls_pad[perm]

    s_pad = jnp.concatenate([scores, jnp.full((pad,), -1.0, jnp.float32)])
    inp = jnp.concatenate(
        [
            reg.T,
            proposals.T,
            scores[None, :],
            classes.astype(jnp.float32)[None, :],
            jnp.arange(n, dtype=jnp.float32)[None, :],
        ],
        axis=0,
    )
    inp = jnp.pad(inp, ((0, 5), (0, 0)))
    inp = jnp.concatenate(
        [inp, jnp.tile(jnp.array([[0.0]] * 8 + [[-1.0], [100.0], [0.0], [0.0], [0.0], [0.0], [0.0], [0.0]], jnp.float32), (1, pad))],
        axis=1,
    )
    inp = inp[:, perm]

    blocks = csort.reshape(n_i, _BR)
    jstart = jnp.searchsorted(csort, blocks[:, 0], side="left")
    jend = jnp.searchsorted(csort, blocks[:, -1], side="right")
    j0 = (jstart // _BC).astype(jnp.int32)
    jn = ((jend + _BC - 1) // _BC).astype(jnp.int32) - j0
    jinfo = jnp.stack([j0, jn], axis=1).reshape(-1)

    stats, boxesT = pl.pallas_call(
        _prep_body,
        out_shape=(
            jax.ShapeDtypeStruct((8, npad), jnp.float32),
            jax.ShapeDtypeStruct((4, npad), jnp.float32),
        ),
    )(inp)

    statsI = stats.T
    stats3 = stats.reshape(8, n_j, _BC).transpose(1, 0, 2)
    supout = pl.pallas_call(
        _sup_body,
        grid_spec=pltpu.PrefetchScalarGridSpec(
            num_scalar_prefetch=1,
            grid=(n_i,),
            in_specs=[
                pl.BlockSpec((_BR, 8), lambda i, jinfo_ref: (i, 0)),
                pl.BlockSpec((n_j, 8, _BC), lambda i, jinfo_ref: (0, 0, 0)),
            ],
            out_specs=pl.BlockSpec((_BR, 8), lambda i, jinfo_ref: (i, 0)),
        ),
        out_shape=jax.ShapeDtypeStruct((npad, 8), jnp.float32),
        compiler_params=pltpu.CompilerParams(
            dimension_semantics=("arbitrary",),
        ),
    )(jinfo, statsI, stats3)

    sup_sorted = supout[:, 0] > 0.5
    valid_sorted = stats[6, :] > 0.5
    suppressed = jnp.zeros((npad,), bool).at[perm].set(sup_sorted)[:n]
    valid = jnp.zeros((npad,), bool).at[perm].set(valid_sorted)[:n]
    keep = valid & (~suppressed)

    sbits = jax.lax.bitcast_convert_type(scores, jnp.int32)
    key = jnp.where(
        keep,
        sbits,
        jnp.where(valid, sbits - jnp.int32(0x3F800001), jnp.int32(-0x7F000000)),
    )
    _, sel = jax.lax.top_k(key, _IMTOP)

    pos = jnp.zeros((npad,), jnp.int32).at[perm].set(jnp.arange(npad, dtype=jnp.int32))
    boxes_sel = boxesT[:, pos[sel]].T
    return boxes_sel, scores[sel], classes[sel]
